# Initial kernel scaffold; baseline (speedup 1.0000x reference)
import functools

import jax
import jax.numpy as jnp
from jax import lax
from jax.experimental import pallas as pl
from jax.experimental.pallas import tpu as pltpu
from jax.experimental.pallas import tpu_sc as plsc

_NC = 2
_NS = 16
_NW = _NC * _NS
_CHUNK = 80
_ZROWS = 25


def _mesh():
    return plsc.VectorSubcoreMesh(core_axis_name="c", subcore_axis_name="s")


@functools.lru_cache(maxsize=None)
def _make_spmm(n, e, f):
    ept = e // _NW
    nch = ept // _CHUNK
    rpt = n // _NS

    @functools.partial(
        pl.kernel,
        out_type=jax.ShapeDtypeStruct((_NC, n, f), jnp.float32),
        mesh=_mesh(),
        scratch_types=[
            pltpu.VMEM((ept,), jnp.int32),
            pltpu.VMEM((nch, _CHUNK), jnp.int32),
            pltpu.VMEM((_CHUNK, f), jnp.float32),
            pltpu.VMEM((_ZROWS, f), jnp.float32),
            pltpu.VMEM_SHARED((n, f), jnp.float32),
            pltpu.SemaphoreType.DMA,
        ],
    )
    def spmm(src_hbm, dst2_hbm, z_hbm, out_hbm,
             src_v, dst_v, rows_v, zero_v, acc_sh, sem):
        c = lax.axis_index("c")
        s = lax.axis_index("s")
        wid = s * _NC + c

        pltpu.sync_copy(src_hbm.at[pl.ds(wid * ept, ept)], src_v)
        pltpu.sync_copy(dst2_hbm.at[pl.ds(wid * nch, nch)], dst_v)

        def zstore(i, _):
            zero_v[i // (f // 16), pl.ds((i % (f // 16)) * 16, 16)] = (
                jnp.zeros((16,), jnp.float32))
            return 0
        lax.fori_loop(0, _ZROWS * (f // 16), zstore, 0)
        base = s * rpt
        for r in range(rpt // _ZROWS):
            pltpu.sync_copy(zero_v, acc_sh.at[pl.ds(base + r * _ZROWS, _ZROWS)])
        plsc.subcore_barrier()

        def body(j, _):
            off = pl.multiple_of(j * _CHUNK, 8)
            pltpu.async_copy(
                z_hbm.at[src_v.at[pl.ds(off, _CHUNK)]], rows_v, sem).wait()
            pltpu.sync_copy(rows_v, acc_sh.at[dst_v.at[j]], add=True)
            return 0
        lax.fori_loop(0, nch, body, 0)
        plsc.subcore_barrier()

        pltpu.sync_copy(acc_sh.at[pl.ds(base, rpt)],
                        out_hbm.at[c, pl.ds(base, rpt)])

    return spmm


@functools.lru_cache(maxsize=None)
def _make_deg(n, e):
    ept = e // _NW
    nch = ept // _CHUNK
    big = 640
    tail = n - big * (_NS - 1)

    @functools.partial(
        pl.kernel,
        out_type=jax.ShapeDtypeStruct((_NC, n), jnp.float32),
        mesh=_mesh(),
        scratch_types=[
            pltpu.VMEM((nch, _CHUNK), jnp.int32),
            pltpu.VMEM((_CHUNK,), jnp.float32),
            pltpu.VMEM((big,), jnp.float32),
            pltpu.VMEM_SHARED((n,), jnp.float32),
        ],
    )
    def deg(dst2_hbm, out_hbm, dst_v, ones_v, zero_v, acc_sh):
        c = lax.axis_index("c")
        s = lax.axis_index("s")
        wid = s * _NC + c

        pltpu.sync_copy(dst2_hbm.at[pl.ds(wid * nch, nch)], dst_v)
        for i in range(_CHUNK // 16):
            ones_v[pl.ds(i * 16, 16)] = jnp.ones((16,), jnp.float32)
        for i in range(big // 16):
            zero_v[pl.ds(i * 16, 16)] = jnp.zeros((16,), jnp.float32)

        @pl.when(s < _NS - 1)
        def _():
            pltpu.sync_copy(zero_v, acc_sh.at[pl.ds(s * big, big)])

        @pl.when(s == _NS - 1)
        def _():
            pltpu.sync_copy(zero_v.at[pl.ds(0, tail)],
                            acc_sh.at[pl.ds((_NS - 1) * big, tail)])
        plsc.subcore_barrier()

        def body(j, _):
            pltpu.sync_copy(ones_v, acc_sh.at[dst_v.at[j]], add=True)
            return 0
        lax.fori_loop(0, nch, body, 0)
        plsc.subcore_barrier()

        @pl.when(s < _NS - 1)
        def _():
            pltpu.sync_copy(acc_sh.at[pl.ds(s * big, big)],
                            out_hbm.at[c, pl.ds(s * big, big)])

        @pl.when(s == _NS - 1)
        def _():
            pltpu.sync_copy(acc_sh.at[pl.ds((_NS - 1) * big, tail)],
                            out_hbm.at[c, pl.ds((_NS - 1) * big, tail)])

    return deg


_BM = 400


@functools.lru_cache(maxsize=None)
def _make_mm(n, k, f):
    def body(h_ref, w_ref, o_ref):
        o_ref[...] = jnp.dot(h_ref[...], w_ref[...],
                             preferred_element_type=jnp.float32)
    return pl.pallas_call(
        body,
        grid=(n // _BM,),
        in_specs=[
            pl.BlockSpec((_BM, k), lambda i: (i, 0)),
            pl.BlockSpec((k, f), lambda i: (0, 0)),
        ],
        out_specs=pl.BlockSpec((_BM, f), lambda i: (i, 0)),
        out_shape=jax.ShapeDtypeStruct((n, f), jnp.float32),
    )


@functools.lru_cache(maxsize=None)
def _make_combine(n, k, f, act):
    def body(h_ref, w_ref, b_ref, s_ref, d_ref, o_ref):
        r = jnp.dot(h_ref[...], w_ref[...], preferred_element_type=jnp.float32)
        dsum = d_ref[0] + d_ref[1]
        inv = 1.0 / jnp.maximum(dsum, 1.0)
        r = r + b_ref[...] + (s_ref[0] + s_ref[1]) * inv
        o_ref[...] = jnp.maximum(r, 0.0) if act else r
    return pl.pallas_call(
        body,
        grid=(n // _BM,),
        in_specs=[
            pl.BlockSpec((_BM, k), lambda i: (i, 0)),
            pl.BlockSpec((k, f), lambda i: (0, 0)),
            pl.BlockSpec((1, f), lambda i: (0, 0)),
            pl.BlockSpec((_NC, _BM, f), lambda i: (0, i, 0)),
            pl.BlockSpec((_NC, _BM, 1), lambda i: (0, i, 0)),
        ],
        out_specs=pl.BlockSpec((_BM, f), lambda i: (i, 0)),
        out_shape=jax.ShapeDtypeStruct((n, f), jnp.float32),
    )


def kernel(x, adj, W1, b1, W2, b2, W3, b3, W4, b4):
    n = x.shape[0]
    e = adj.shape[1]
    src = adj[0]
    dst2 = adj[1].reshape(e // _CHUNK, _CHUNK)

    deg2 = _make_deg(n, e)(dst2)
    deg3 = deg2.reshape(_NC, n, 1)

    h = x
    for W, b, act in ((W1, b1, True), (W2, b2, True),
                      (W3, b3, True), (W4, b4, False)):
        k = W.shape[0] // 2
        f = W.shape[1]
        z = _make_mm(n, k, f)(h, W[k:])
        s2 = _make_spmm(n, e, f)(src, dst2, z)
        h = _make_combine(n, k, f, act)(
            h, W[:k], b.reshape(1, f), s2, deg3)
    return h


# speedup vs baseline: 6.1682x; 6.1682x over previous
import functools

import jax
import jax.numpy as jnp
from jax import lax
from jax.experimental import pallas as pl
from jax.experimental.pallas import tpu as pltpu
from jax.experimental.pallas import tpu_sc as plsc

_NC = 2
_NS = 16
_NW = _NC * _NS
_CHUNK = 80
_ZROWS = 80
_RPT = 640
_NPAD = _NS * _RPT
_DEGPT = 1024
_DEGPAD = _NS * _DEGPT


def _mesh():
    return plsc.VectorSubcoreMesh(core_axis_name="c", subcore_axis_name="s")


@functools.lru_cache(maxsize=None)
def _make_spmm(n, e, f):
    ept = e // _NW
    nch = ept // _CHUNK

    @functools.partial(
        pl.kernel,
        out_type=jax.ShapeDtypeStruct((_NC, _NPAD, f), jnp.float32),
        mesh=_mesh(),
        scratch_types=[
            pltpu.VMEM((nch, _CHUNK), jnp.int32),
            pltpu.VMEM((nch, _CHUNK), jnp.int32),
            pltpu.VMEM((_CHUNK, f), jnp.float32),
            pltpu.VMEM_SHARED((_NPAD, f), jnp.float32),
            pltpu.SemaphoreType.DMA,
        ],
    )
    def spmm(src3_hbm, dst3_hbm, z_hbm, out_hbm,
             src_v, dst_v, rows_v, acc_sh, sem):
        c = lax.axis_index("c")
        s = lax.axis_index("s")
        wid = s * _NC + c

        pltpu.sync_copy(src3_hbm.at[wid], src_v)
        pltpu.sync_copy(dst3_hbm.at[wid], dst_v)

        def zstore(i, _):
            rows_v[i // (f // 16), pl.ds((i % (f // 16)) * 16, 16)] = (
                jnp.zeros((16,), jnp.float32))
            return 0
        lax.fori_loop(0, _CHUNK * (f // 16), zstore, 0)
        base = s * _RPT
        for r in range(_RPT // _CHUNK):
            pltpu.sync_copy(rows_v, acc_sh.at[pl.ds(base + r * _CHUNK, _CHUNK)])
        plsc.subcore_barrier()

        def body(j, _):
            pltpu.async_copy(z_hbm.at[src_v.at[j]], rows_v, sem).wait()
            pltpu.sync_copy(rows_v, acc_sh.at[dst_v.at[j]], add=True)
            return 0
        lax.fori_loop(0, nch, body, 0)
        plsc.subcore_barrier()

        pltpu.sync_copy(acc_sh.at[pl.ds(base, _RPT)],
                        out_hbm.at[c, pl.ds(base, _RPT)])

    return spmm


@functools.lru_cache(maxsize=None)
def _make_deg(n, e):
    ept = e // _NW
    nch = ept // _CHUNK

    @functools.partial(
        pl.kernel,
        out_type=jax.ShapeDtypeStruct((_NC * _DEGPAD,), jnp.float32),
        mesh=_mesh(),
        scratch_types=[
            pltpu.VMEM((nch, _CHUNK), jnp.int32),
            pltpu.VMEM((_CHUNK,), jnp.float32),
            pltpu.VMEM((_DEGPT,), jnp.float32),
            pltpu.VMEM_SHARED((_DEGPAD,), jnp.float32),
        ],
    )
    def deg(dst3_hbm, out_hbm, dst_v, ones_v, zero_v, acc_sh):
        c = lax.axis_index("c")
        s = lax.axis_index("s")
        wid = s * _NC + c

        pltpu.sync_copy(dst3_hbm.at[wid], dst_v)
        for i in range(_CHUNK // 16):
            ones_v[pl.ds(i * 16, 16)] = jnp.ones((16,), jnp.float32)

        def zstore(i, _):
            zero_v[pl.ds(i * 16, 16)] = jnp.zeros((16,), jnp.float32)
            return 0
        lax.fori_loop(0, _DEGPT // 16, zstore, 0)
        pltpu.sync_copy(zero_v, acc_sh.at[pl.ds(s * _DEGPT, _DEGPT)])
        plsc.subcore_barrier()

        def body(j, _):
            pltpu.sync_copy(ones_v, acc_sh.at[dst_v.at[j]], add=True)
            return 0
        lax.fori_loop(0, nch, body, 0)
        plsc.subcore_barrier()

        pltpu.sync_copy(acc_sh.at[pl.ds(s * _DEGPT, _DEGPT)],
                        out_hbm.at[pl.ds(c * _DEGPAD + s * _DEGPT, _DEGPT)])

    return deg


_BM = 400


@functools.lru_cache(maxsize=None)
def _make_mm(n, k, f, fpad):
    def body(h_ref, w_ref, o_ref):
        r = jnp.dot(h_ref[...], w_ref[...], preferred_element_type=jnp.float32)
        if fpad > f:
            r = jnp.concatenate(
                [r, jnp.zeros((_BM, fpad - f), jnp.float32)], axis=1)
        o_ref[...] = r
    return pl.pallas_call(
        body,
        grid=(n // _BM,),
        in_specs=[
            pl.BlockSpec((_BM, k), lambda i: (i, 0)),
            pl.BlockSpec((k, f), lambda i: (0, 0)),
        ],
        out_specs=pl.BlockSpec((_BM, fpad), lambda i: (i, 0)),
        out_shape=jax.ShapeDtypeStruct((n, fpad), jnp.float32),
    )


@functools.lru_cache(maxsize=None)
def _make_combine(n, k, f, fpad, act):
    def body(h_ref, w_ref, b_ref, s_ref, d_ref, o_ref):
        r = jnp.dot(h_ref[...], w_ref[...], preferred_element_type=jnp.float32)
        dsum = d_ref[0] + d_ref[1]
        inv = 1.0 / jnp.maximum(dsum, 1.0)
        r = r + b_ref[...] + (s_ref[0, :, :f] + s_ref[1, :, :f]) * inv
        o_ref[...] = jnp.maximum(r, 0.0) if act else r
    return pl.pallas_call(
        body,
        grid=(n // _BM,),
        in_specs=[
            pl.BlockSpec((_BM, k), lambda i: (i, 0)),
            pl.BlockSpec((k, f), lambda i: (0, 0)),
            pl.BlockSpec((1, f), lambda i: (0, 0)),
            pl.BlockSpec((_NC, _BM, fpad), lambda i: (0, i, 0)),
            pl.BlockSpec((_NC, _BM, 1), lambda i: (0, i, 0)),
        ],
        out_specs=pl.BlockSpec((_BM, f), lambda i: (i, 0)),
        out_shape=jax.ShapeDtypeStruct((n, f), jnp.float32),
    )


def kernel(x, adj, W1, b1, W2, b2, W3, b3, W4, b4):
    n = x.shape[0]
    e = adj.shape[1]
    nch = e // (_NW * _CHUNK)
    src3 = adj[0].reshape(_NW, nch, _CHUNK)
    dst3 = adj[1].reshape(_NW, nch, _CHUNK)

    deg_flat = _make_deg(n, e)(dst3)
    deg3 = deg_flat.reshape(_NC, _DEGPAD, 1)

    h = x
    for W, b, act in ((W1, b1, True), (W2, b2, True),
                      (W3, b3, True), (W4, b4, False)):
        k = W.shape[0] // 2
        f = W.shape[1]
        fpad = max(f, 128)
        z = _make_mm(n, k, f, fpad)(h, W[k:])
        s2 = _make_spmm(n, e, fpad)(src3, dst3, z)
        h = _make_combine(n, k, f, fpad, act)(
            h, W[:k], b.reshape(1, f), s2, deg3)
    return h


# speedup vs baseline: 8.1465x; 1.3207x over previous
import functools

import jax
import jax.numpy as jnp
from jax import lax
from jax.experimental import pallas as pl
from jax.experimental.pallas import tpu as pltpu
from jax.experimental.pallas import tpu_sc as plsc

_NC = 2
_NS = 16
_NW = _NC * _NS
_CHUNK = 125
_G = 16
_DCHUNK = 80
_RPT = 640
_NPAD = _NS * _RPT
_DEGPT = 1024
_DEGPAD = _NS * _DEGPT


def _mesh():
    return plsc.VectorSubcoreMesh(core_axis_name="c", subcore_axis_name="s")


@functools.lru_cache(maxsize=None)
def _make_spmm(n, e, f):
    ept = e // _NW
    nch = ept // _CHUNK

    ngrp = nch // _G

    @functools.partial(
        pl.kernel,
        out_type=jax.ShapeDtypeStruct((_NC, _NPAD, f), jnp.float32),
        mesh=_mesh(),
        scratch_types=[
            pltpu.VMEM((_G, _CHUNK), jnp.int32),
            pltpu.VMEM((_G, _CHUNK), jnp.int32),
            pltpu.VMEM((_G, _CHUNK), jnp.int32),
            pltpu.VMEM((_G, _CHUNK), jnp.int32),
            pltpu.VMEM((_CHUNK, f), jnp.float32),
            pltpu.VMEM((_CHUNK, f), jnp.float32),
            pltpu.VMEM_SHARED((_NPAD, f), jnp.float32),
            pltpu.SemaphoreType.DMA,
            pltpu.SemaphoreType.DMA,
            pltpu.SemaphoreType.DMA,
            pltpu.SemaphoreType.DMA,
            pltpu.SemaphoreType.DMA,
            pltpu.SemaphoreType.DMA,
        ],
    )
    def spmm(src3_hbm, dst3_hbm, z_hbm, out_hbm,
             sb0, sb1, db0, db1, rows0, rows1, acc_sh,
             g0, g1, s0, s1, is0, is1):
        c = lax.axis_index("c")
        s = lax.axis_index("s")
        wid = s * _NC + c
        sbufs = (sb0, sb1)
        dbufs = (db0, db1)
        isems = (is0, is1)

        def idx_start(g):
            pltpu.async_copy(src3_hbm.at[wid, pl.ds(g * _G, _G)],
                             sbufs[g % 2], isems[g % 2])
            pltpu.async_copy(dst3_hbm.at[wid, pl.ds(g * _G, _G)],
                             dbufs[g % 2], isems[g % 2])

        def idx_wait(g):
            pltpu.make_async_copy(src3_hbm.at[wid, pl.ds(g * _G, _G)],
                                  sbufs[g % 2], isems[g % 2]).wait()
            pltpu.make_async_copy(dst3_hbm.at[wid, pl.ds(g * _G, _G)],
                                  dbufs[g % 2], isems[g % 2]).wait()

        idx_start(0)

        def zstore(i, _):
            rows0[i // (f // 16), pl.ds((i % (f // 16)) * 16, 16)] = (
                jnp.zeros((16,), jnp.float32))
            return 0
        lax.fori_loop(0, _CHUNK * (f // 16), zstore, 0)
        base = s * _RPT
        for r in range(_RPT // _CHUNK):
            pltpu.sync_copy(rows0, acc_sh.at[pl.ds(base + r * _CHUNK, _CHUNK)])
        rem = _RPT - (_RPT // _CHUNK) * _CHUNK
        if rem:
            pltpu.sync_copy(
                rows0.at[pl.ds(0, rem)],
                acc_sh.at[pl.ds(base + (_RPT // _CHUNK) * _CHUNK, rem)])
        plsc.subcore_barrier()

        def wait_g(sbuf, k, buf, sem):
            pltpu.make_async_copy(z_hbm.at[sbuf.at[k]], buf, sem).wait()

        idx_wait(0)
        pltpu.async_copy(z_hbm.at[sb0.at[0]], rows0, g0)
        pltpu.async_copy(z_hbm.at[sb0.at[1]], rows1, g1)

        for g in range(ngrp):
            sbuf, dbuf = sbufs[g % 2], dbufs[g % 2]
            if g + 1 < ngrp:
                idx_start(g + 1)

            def pair(p, _, sbuf=sbuf, dbuf=dbuf):
                k = p * 2
                wait_g(sbuf, k, rows0, g0)
                d0 = pltpu.async_copy(rows0, acc_sh.at[dbuf.at[k]], s0,
                                      add=True)
                wait_g(sbuf, k + 1, rows1, g1)
                d1 = pltpu.async_copy(rows1, acc_sh.at[dbuf.at[k + 1]], s1,
                                      add=True)
                d0.wait()
                pltpu.async_copy(z_hbm.at[sbuf.at[k + 2]], rows0, g0)
                d1.wait()
                pltpu.async_copy(z_hbm.at[sbuf.at[k + 3]], rows1, g1)
                return 0
            lax.fori_loop(0, _G // 2 - 1, pair, 0)

            k = _G - 2
            wait_g(sbuf, k, rows0, g0)
            d0 = pltpu.async_copy(rows0, acc_sh.at[dbuf.at[k]], s0, add=True)
            wait_g(sbuf, k + 1, rows1, g1)
            d1 = pltpu.async_copy(rows1, acc_sh.at[dbuf.at[k + 1]], s1,
                                  add=True)
            if g + 1 < ngrp:
                idx_wait(g + 1)
                d0.wait()
                pltpu.async_copy(z_hbm.at[sbufs[(g + 1) % 2].at[0]], rows0, g0)
                d1.wait()
                pltpu.async_copy(z_hbm.at[sbufs[(g + 1) % 2].at[1]], rows1, g1)
            else:
                d0.wait()
                d1.wait()
        plsc.subcore_barrier()

        pltpu.sync_copy(acc_sh.at[pl.ds(base, _RPT)],
                        out_hbm.at[c, pl.ds(base, _RPT)])

    return spmm


@functools.lru_cache(maxsize=None)
def _make_deg(n, e):
    ept = e // _NW
    nch = ept // _DCHUNK

    @functools.partial(
        pl.kernel,
        out_type=jax.ShapeDtypeStruct((_NC * _DEGPAD,), jnp.float32),
        mesh=_mesh(),
        scratch_types=[
            pltpu.VMEM((nch, _DCHUNK), jnp.int32),
            pltpu.VMEM((_DCHUNK,), jnp.float32),
            pltpu.VMEM((_DEGPT,), jnp.float32),
            pltpu.VMEM_SHARED((_DEGPAD,), jnp.float32),
        ],
    )
    def deg(dst3_hbm, out_hbm, dst_v, ones_v, zero_v, acc_sh):
        c = lax.axis_index("c")
        s = lax.axis_index("s")
        wid = s * _NC + c

        pltpu.sync_copy(dst3_hbm.at[wid], dst_v)
        for i in range(_DCHUNK // 16):
            ones_v[pl.ds(i * 16, 16)] = jnp.ones((16,), jnp.float32)

        def zstore(i, _):
            zero_v[pl.ds(i * 16, 16)] = jnp.zeros((16,), jnp.float32)
            return 0
        lax.fori_loop(0, _DEGPT // 16, zstore, 0)
        pltpu.sync_copy(zero_v, acc_sh.at[pl.ds(s * _DEGPT, _DEGPT)])
        plsc.subcore_barrier()

        def body(j, _):
            pltpu.sync_copy(ones_v, acc_sh.at[dst_v.at[j]], add=True)
            return 0
        lax.fori_loop(0, nch, body, 0)
        plsc.subcore_barrier()

        pltpu.sync_copy(acc_sh.at[pl.ds(s * _DEGPT, _DEGPT)],
                        out_hbm.at[pl.ds(c * _DEGPAD + s * _DEGPT, _DEGPT)])

    return deg


_BM = 400


@functools.lru_cache(maxsize=None)
def _make_mm(n, k, f, fpad):
    def body(h_ref, w_ref, o_ref):
        r = jnp.dot(h_ref[...], w_ref[...], preferred_element_type=jnp.float32)
        if fpad > f:
            r = jnp.concatenate(
                [r, jnp.zeros((_BM, fpad - f), jnp.float32)], axis=1)
        o_ref[...] = r
    return pl.pallas_call(
        body,
        grid=(n // _BM,),
        in_specs=[
            pl.BlockSpec((_BM, k), lambda i: (i, 0)),
            pl.BlockSpec((k, f), lambda i: (0, 0)),
        ],
        out_specs=pl.BlockSpec((_BM, fpad), lambda i: (i, 0)),
        out_shape=jax.ShapeDtypeStruct((n, fpad), jnp.float32),
    )


@functools.lru_cache(maxsize=None)
def _make_combine(n, k, f, fpad, act):
    def body(h_ref, w_ref, b_ref, s_ref, d_ref, o_ref):
        r = jnp.dot(h_ref[...], w_ref[...], preferred_element_type=jnp.float32)
        dsum = d_ref[0] + d_ref[1]
        inv = 1.0 / jnp.maximum(dsum, 1.0)
        r = r + b_ref[...] + (s_ref[0, :, :f] + s_ref[1, :, :f]) * inv
        o_ref[...] = jnp.maximum(r, 0.0) if act else r
    return pl.pallas_call(
        body,
        grid=(n // _BM,),
        in_specs=[
            pl.BlockSpec((_BM, k), lambda i: (i, 0)),
            pl.BlockSpec((k, f), lambda i: (0, 0)),
            pl.BlockSpec((1, f), lambda i: (0, 0)),
            pl.BlockSpec((_NC, _BM, fpad), lambda i: (0, i, 0)),
            pl.BlockSpec((_NC, _BM, 1), lambda i: (0, i, 0)),
        ],
        out_specs=pl.BlockSpec((_BM, f), lambda i: (i, 0)),
        out_shape=jax.ShapeDtypeStruct((n, f), jnp.float32),
    )


def kernel(x, adj, W1, b1, W2, b2, W3, b3, W4, b4):
    n = x.shape[0]
    e = adj.shape[1]
    nch = e // (_NW * _CHUNK)
    src3 = adj[0].reshape(_NW, nch, _CHUNK)
    dst3 = adj[1].reshape(_NW, nch, _CHUNK)
    dst3d = adj[1].reshape(_NW, e // (_NW * _DCHUNK), _DCHUNK)

    deg_flat = _make_deg(n, e)(dst3d)
    deg3 = deg_flat.reshape(_NC, _DEGPAD, 1)

    h = x
    for W, b, act in ((W1, b1, True), (W2, b2, True),
                      (W3, b3, True), (W4, b4, False)):
        k = W.shape[0] // 2
        f = W.shape[1]
        fpad = max(f, 128)
        z = _make_mm(n, k, f, fpad)(h, W[k:])
        s2 = _make_spmm(n, e, fpad)(src3, dst3, z)
        h = _make_combine(n, k, f, fpad, act)(
            h, W[:k], b.reshape(1, f), s2, deg3)
    return h


# speedup vs baseline: 8.6003x; 1.0557x over previous
import functools

import jax
import jax.numpy as jnp
from jax import lax
from jax.experimental import pallas as pl
from jax.experimental.pallas import tpu as pltpu
from jax.experimental.pallas import tpu_sc as plsc

_NC = 2
_NS = 16
_NW = _NC * _NS
_CHUNK = 125
_G = 16
_DCHUNK = 80
_RPT = 640
_NPAD = _NS * _RPT
_DEGPT = 1024
_DEGPAD = _NS * _DEGPT


def _mesh():
    return plsc.VectorSubcoreMesh(core_axis_name="c", subcore_axis_name="s")


@functools.lru_cache(maxsize=None)
def _make_spmm(n, e, f):
    ept = e // _NW
    nch = ept // _CHUNK

    ngrp = nch // _G

    @functools.partial(
        pl.kernel,
        out_type=jax.ShapeDtypeStruct((_NC, _NPAD, f), jnp.float32),
        mesh=_mesh(),
        scratch_types=[
            pltpu.VMEM((_G, _CHUNK), jnp.int32),
            pltpu.VMEM((_G, _CHUNK), jnp.int32),
            pltpu.VMEM((_G, _CHUNK), jnp.int32),
            pltpu.VMEM((_G, _CHUNK), jnp.int32),
            pltpu.VMEM((_CHUNK, f), jnp.float32),
            pltpu.VMEM((_CHUNK, f), jnp.float32),
            pltpu.VMEM_SHARED((_NPAD, f), jnp.float32),
            pltpu.SemaphoreType.DMA,
            pltpu.SemaphoreType.DMA,
            pltpu.SemaphoreType.DMA,
            pltpu.SemaphoreType.DMA,
            pltpu.SemaphoreType.DMA,
            pltpu.SemaphoreType.DMA,
        ],
    )
    def spmm(src3_hbm, dst3_hbm, z_hbm, out_hbm,
             sb0, sb1, db0, db1, rows0, rows1, acc_sh,
             g0, g1, s0, s1, is0, is1):
        c = lax.axis_index("c")
        s = lax.axis_index("s")
        wid = s * _NC + c
        sbufs = (sb0, sb1)
        dbufs = (db0, db1)
        isems = (is0, is1)

        def idx_start(g):
            pltpu.async_copy(src3_hbm.at[wid, pl.ds(g * _G, _G)],
                             sbufs[g % 2], isems[g % 2])
            pltpu.async_copy(dst3_hbm.at[wid, pl.ds(g * _G, _G)],
                             dbufs[g % 2], isems[g % 2])

        def idx_wait(g):
            pltpu.make_async_copy(src3_hbm.at[wid, pl.ds(g * _G, _G)],
                                  sbufs[g % 2], isems[g % 2]).wait()
            pltpu.make_async_copy(dst3_hbm.at[wid, pl.ds(g * _G, _G)],
                                  dbufs[g % 2], isems[g % 2]).wait()

        idx_start(0)

        def zstore(i, _):
            rows0[i // (f // 16), pl.ds((i % (f // 16)) * 16, 16)] = (
                jnp.zeros((16,), jnp.float32))
            return 0
        lax.fori_loop(0, _CHUNK * (f // 16), zstore, 0)
        base = s * _RPT
        for r in range(_RPT // _CHUNK):
            pltpu.sync_copy(rows0, acc_sh.at[pl.ds(base + r * _CHUNK, _CHUNK)])
        rem = _RPT - (_RPT // _CHUNK) * _CHUNK
        if rem:
            pltpu.sync_copy(
                rows0.at[pl.ds(0, rem)],
                acc_sh.at[pl.ds(base + (_RPT // _CHUNK) * _CHUNK, rem)])
        plsc.subcore_barrier()

        def wait_g(sbuf, k, buf, sem):
            pltpu.make_async_copy(z_hbm.at[sbuf.at[k]], buf, sem).wait()

        idx_wait(0)
        pltpu.async_copy(z_hbm.at[sb0.at[0]], rows0, g0)
        pltpu.async_copy(z_hbm.at[sb0.at[1]], rows1, g1)

        for g in range(ngrp):
            sbuf, dbuf = sbufs[g % 2], dbufs[g % 2]
            if g + 1 < ngrp:
                idx_start(g + 1)

            def pair(p, _, sbuf=sbuf, dbuf=dbuf):
                k = p * 2
                wait_g(sbuf, k, rows0, g0)
                d0 = pltpu.async_copy(rows0, acc_sh.at[dbuf.at[k]], s0,
                                      add=True)
                wait_g(sbuf, k + 1, rows1, g1)
                d1 = pltpu.async_copy(rows1, acc_sh.at[dbuf.at[k + 1]], s1,
                                      add=True)
                d0.wait()
                pltpu.async_copy(z_hbm.at[sbuf.at[k + 2]], rows0, g0)
                d1.wait()
                pltpu.async_copy(z_hbm.at[sbuf.at[k + 3]], rows1, g1)
                return 0
            lax.fori_loop(0, _G // 2 - 1, pair, 0)

            k = _G - 2
            wait_g(sbuf, k, rows0, g0)
            d0 = pltpu.async_copy(rows0, acc_sh.at[dbuf.at[k]], s0, add=True)
            wait_g(sbuf, k + 1, rows1, g1)
            d1 = pltpu.async_copy(rows1, acc_sh.at[dbuf.at[k + 1]], s1,
                                  add=True)
            if g + 1 < ngrp:
                idx_wait(g + 1)
                d0.wait()
                pltpu.async_copy(z_hbm.at[sbufs[(g + 1) % 2].at[0]], rows0, g0)
                d1.wait()
                pltpu.async_copy(z_hbm.at[sbufs[(g + 1) % 2].at[1]], rows1, g1)
            else:
                d0.wait()
                d1.wait()
        plsc.subcore_barrier()

        pltpu.sync_copy(acc_sh.at[pl.ds(base, _RPT)],
                        out_hbm.at[c, pl.ds(base, _RPT)])

    return spmm


@functools.lru_cache(maxsize=None)
def _make_deg(n, e):
    ept = e // _NW
    nch = ept // _DCHUNK

    @functools.partial(
        pl.kernel,
        out_type=jax.ShapeDtypeStruct((_NC * _DEGPAD,), jnp.float32),
        mesh=_mesh(),
        scratch_types=[
            pltpu.VMEM((nch, _DCHUNK), jnp.int32),
            pltpu.VMEM((_DCHUNK,), jnp.float32),
            pltpu.VMEM((_DEGPT,), jnp.float32),
            pltpu.VMEM_SHARED((_DEGPAD,), jnp.float32),
        ],
    )
    def deg(dst3_hbm, out_hbm, dst_v, ones_v, zero_v, acc_sh):
        c = lax.axis_index("c")
        s = lax.axis_index("s")
        wid = s * _NC + c

        pltpu.sync_copy(dst3_hbm.at[wid], dst_v)
        for i in range(_DCHUNK // 16):
            ones_v[pl.ds(i * 16, 16)] = jnp.ones((16,), jnp.float32)

        def zstore(i, _):
            zero_v[pl.ds(i * 16, 16)] = jnp.zeros((16,), jnp.float32)
            return 0
        lax.fori_loop(0, _DEGPT // 16, zstore, 0)
        pltpu.sync_copy(zero_v, acc_sh.at[pl.ds(s * _DEGPT, _DEGPT)])
        plsc.subcore_barrier()

        def body(j, _):
            pltpu.sync_copy(ones_v, acc_sh.at[dst_v.at[j]], add=True)
            return 0
        lax.fori_loop(0, nch, body, 0)
        plsc.subcore_barrier()

        pltpu.sync_copy(acc_sh.at[pl.ds(s * _DEGPT, _DEGPT)],
                        out_hbm.at[pl.ds(c * _DEGPAD + s * _DEGPT, _DEGPT)])

    return deg


_BM = 400


@functools.lru_cache(maxsize=None)
def _make_mm(n, k, f, fpad):
    def body(h_ref, w_ref, o_ref):
        r = jnp.dot(h_ref[...], w_ref[...], preferred_element_type=jnp.float32)
        if fpad > f:
            r = jnp.concatenate(
                [r, jnp.zeros((_BM, fpad - f), jnp.float32)], axis=1)
        o_ref[...] = r
    return pl.pallas_call(
        body,
        grid=(n // _BM,),
        in_specs=[
            pl.BlockSpec((_BM, k), lambda i: (i, 0)),
            pl.BlockSpec((k, f), lambda i: (0, 0)),
        ],
        out_specs=pl.BlockSpec((_BM, fpad), lambda i: (i, 0)),
        out_shape=jax.ShapeDtypeStruct((n, fpad), jnp.float32),
    )


@functools.lru_cache(maxsize=None)
def _make_fused(n, k, f, f_next, fpad_next):
    fpad = max(f, 128)
    def body(h_ref, w_ref, b_ref, s_ref, d_ref, wn_ref, o_ref, z_ref):
        r = jnp.dot(h_ref[...], w_ref[...], preferred_element_type=jnp.float32)
        dsum = d_ref[0] + d_ref[1]
        inv = 1.0 / jnp.maximum(dsum, 1.0)
        r = r + b_ref[...] + (s_ref[0, :, :f] + s_ref[1, :, :f]) * inv
        hn = jnp.maximum(r, 0.0)
        o_ref[...] = hn
        z = jnp.dot(hn, wn_ref[...], preferred_element_type=jnp.float32)
        if fpad_next > f_next:
            z = jnp.concatenate(
                [z, jnp.zeros((_BM, fpad_next - f_next), jnp.float32)], axis=1)
        z_ref[...] = z
    return pl.pallas_call(
        body,
        grid=(n // _BM,),
        in_specs=[
            pl.BlockSpec((_BM, k), lambda i: (i, 0)),
            pl.BlockSpec((k, f), lambda i: (0, 0)),
            pl.BlockSpec((1, f), lambda i: (0, 0)),
            pl.BlockSpec((_NC, _BM, fpad), lambda i: (0, i, 0)),
            pl.BlockSpec((_NC, _BM, 1), lambda i: (0, i, 0)),
            pl.BlockSpec((f, f_next), lambda i: (0, 0)),
        ],
        out_specs=[
            pl.BlockSpec((_BM, f), lambda i: (i, 0)),
            pl.BlockSpec((_BM, fpad_next), lambda i: (i, 0)),
        ],
        out_shape=[
            jax.ShapeDtypeStruct((n, f), jnp.float32),
            jax.ShapeDtypeStruct((n, fpad_next), jnp.float32),
        ],
    )


@functools.lru_cache(maxsize=None)
def _make_combine(n, k, f, fpad, act):
    def body(h_ref, w_ref, b_ref, s_ref, d_ref, o_ref):
        r = jnp.dot(h_ref[...], w_ref[...], preferred_element_type=jnp.float32)
        dsum = d_ref[0] + d_ref[1]
        inv = 1.0 / jnp.maximum(dsum, 1.0)
        r = r + b_ref[...] + (s_ref[0, :, :f] + s_ref[1, :, :f]) * inv
        o_ref[...] = jnp.maximum(r, 0.0) if act else r
    return pl.pallas_call(
        body,
        grid=(n // _BM,),
        in_specs=[
            pl.BlockSpec((_BM, k), lambda i: (i, 0)),
            pl.BlockSpec((k, f), lambda i: (0, 0)),
            pl.BlockSpec((1, f), lambda i: (0, 0)),
            pl.BlockSpec((_NC, _BM, fpad), lambda i: (0, i, 0)),
            pl.BlockSpec((_NC, _BM, 1), lambda i: (0, i, 0)),
        ],
        out_specs=pl.BlockSpec((_BM, f), lambda i: (i, 0)),
        out_shape=jax.ShapeDtypeStruct((n, f), jnp.float32),
    )


def kernel(x, adj, W1, b1, W2, b2, W3, b3, W4, b4):
    n = x.shape[0]
    e = adj.shape[1]
    nch = e // (_NW * _CHUNK)
    src3 = adj[0].reshape(_NW, nch, _CHUNK)
    dst3 = adj[1].reshape(_NW, nch, _CHUNK)
    dst3d = adj[1].reshape(_NW, e // (_NW * _DCHUNK), _DCHUNK)

    deg_flat = _make_deg(n, e)(dst3d)
    deg3 = deg_flat.reshape(_NC, _DEGPAD, 1)

    Ws = (W1, W2, W3, W4)
    bs = (b1, b2, b3, b4)
    ks = [W.shape[0] // 2 for W in Ws]
    fs = [W.shape[1] for W in Ws]
    fpads = [max(f, 128) for f in fs]

    h = x
    z = _make_mm(n, ks[0], fs[0], fpads[0])(h, W1[ks[0]:])
    for i in range(3):
        s2 = _make_spmm(n, e, fpads[i])(src3, dst3, z)
        h, z = _make_fused(n, ks[i], fs[i], fs[i + 1], fpads[i + 1])(
            h, Ws[i][:ks[i]], bs[i].reshape(1, fs[i]), s2, deg3,
            Ws[i + 1][ks[i + 1]:])
    s2 = _make_spmm(n, e, fpads[3])(src3, dst3, z)
    return _make_combine(n, ks[3], fs[3], fpads[3], False)(
        h, W4[:ks[3]], b4.reshape(1, fs[3]), s2, deg3)


# speedup vs baseline: 8.6633x; 1.0073x over previous
import functools

import jax
import jax.numpy as jnp
from jax import lax
from jax.experimental import pallas as pl
from jax.experimental.pallas import tpu as pltpu
from jax.experimental.pallas import tpu_sc as plsc

_NC = 2
_NS = 16
_NW = _NC * _NS
_CHUNK = 125
_G = 16
_RPT = 640
_NPAD = _NS * _RPT
_DEGPT = 1024
_DEGPAD = _NS * _DEGPT


def _mesh():
    return plsc.VectorSubcoreMesh(core_axis_name="c", subcore_axis_name="s")


@functools.lru_cache(maxsize=None)
def _make_spmm(n, e, f, with_deg):
    ept = e // _NW
    nch = ept // _CHUNK
    ngrp = nch // _G

    out_type = jax.ShapeDtypeStruct((_NC, _NPAD, f), jnp.float32)
    scratch = [
        pltpu.VMEM((_G, _CHUNK), jnp.int32),
        pltpu.VMEM((_G, _CHUNK), jnp.int32),
        pltpu.VMEM((_G, _CHUNK), jnp.int32),
        pltpu.VMEM((_G, _CHUNK), jnp.int32),
        pltpu.VMEM((_CHUNK, f), jnp.float32),
        pltpu.VMEM((_CHUNK, f), jnp.float32),
        pltpu.VMEM_SHARED((_NPAD, f), jnp.float32),
        pltpu.SemaphoreType.DMA,
        pltpu.SemaphoreType.DMA,
        pltpu.SemaphoreType.DMA,
        pltpu.SemaphoreType.DMA,
        pltpu.SemaphoreType.DMA,
        pltpu.SemaphoreType.DMA,
    ]
    if with_deg:
        out_type = (out_type,
                    jax.ShapeDtypeStruct((_NC * _DEGPAD,), jnp.float32))
        scratch = scratch + [
            pltpu.VMEM((128,), jnp.float32),
            pltpu.VMEM((_DEGPT,), jnp.float32),
            pltpu.VMEM_SHARED((_DEGPAD,), jnp.float32),
        ]

    @functools.partial(
        pl.kernel, out_type=out_type, mesh=_mesh(), scratch_types=scratch)
    def spmm(src3_hbm, dst3_hbm, z_hbm, *refs):
        if with_deg:
            (out_hbm, deg_hbm, sb0, sb1, db0, db1, rows0, rows1, acc_sh,
             g0, g1, s0, s1, is0, is1, ones_v, dz_v, dacc_sh) = refs
        else:
            (out_hbm, sb0, sb1, db0, db1, rows0, rows1, acc_sh,
             g0, g1, s0, s1, is0, is1) = refs
        c = lax.axis_index("c")
        s = lax.axis_index("s")
        wid = s * _NC + c
        sbufs = (sb0, sb1)
        dbufs = (db0, db1)
        isems = (is0, is1)

        def idx_start(g):
            pltpu.async_copy(src3_hbm.at[wid, pl.ds(g * _G, _G)],
                             sbufs[g % 2], isems[g % 2])
            pltpu.async_copy(dst3_hbm.at[wid, pl.ds(g * _G, _G)],
                             dbufs[g % 2], isems[g % 2])

        def idx_wait(g):
            pltpu.make_async_copy(src3_hbm.at[wid, pl.ds(g * _G, _G)],
                                  sbufs[g % 2], isems[g % 2]).wait()
            pltpu.make_async_copy(dst3_hbm.at[wid, pl.ds(g * _G, _G)],
                                  dbufs[g % 2], isems[g % 2]).wait()

        idx_start(0)

        def zstore(i, _):
            rows0[i // (f // 16), pl.ds((i % (f // 16)) * 16, 16)] = (
                jnp.zeros((16,), jnp.float32))
            return 0
        lax.fori_loop(0, _CHUNK * (f // 16), zstore, 0)
        base = s * _RPT
        zds = []
        for r in range(_RPT // _CHUNK):
            zds.append(pltpu.async_copy(
                rows0, acc_sh.at[pl.ds(base + r * _CHUNK, _CHUNK)], s0))
        rem = _RPT - (_RPT // _CHUNK) * _CHUNK
        if rem:
            zds.append(pltpu.async_copy(
                rows0.at[pl.ds(0, rem)],
                acc_sh.at[pl.ds(base + (_RPT // _CHUNK) * _CHUNK, rem)], s0))
        if with_deg:
            for i in range(128 // 16):
                ones_v[pl.ds(i * 16, 16)] = jnp.ones((16,), jnp.float32)

            def dzstore(i, _):
                dz_v[pl.ds(i * 16, 16)] = jnp.zeros((16,), jnp.float32)
                return 0
            lax.fori_loop(0, _DEGPT // 16, dzstore, 0)
            zds.append(pltpu.async_copy(
                dz_v, dacc_sh.at[pl.ds(s * _DEGPT, _DEGPT)], s1))
        for d in zds:
            d.wait()
        plsc.subcore_barrier()

        def wait_g(sbuf, k, buf, sem):
            pltpu.make_async_copy(z_hbm.at[sbuf.at[k]], buf, sem).wait()

        def scat(buf, dbuf, k, sem):
            d = pltpu.async_copy(buf, acc_sh.at[dbuf.at[k]], sem, add=True)
            if with_deg:
                dd = pltpu.async_copy(ones_v.at[pl.ds(0, _CHUNK)],
                                      dacc_sh.at[dbuf.at[k]], sem, add=True)
                return (d, dd)
            return (d,)

        idx_wait(0)
        pltpu.async_copy(z_hbm.at[sb0.at[0]], rows0, g0)
        pltpu.async_copy(z_hbm.at[sb0.at[1]], rows1, g1)

        for g in range(ngrp):
            sbuf, dbuf = sbufs[g % 2], dbufs[g % 2]
            if g + 1 < ngrp:
                idx_start(g + 1)

            def pair(p, _, sbuf=sbuf, dbuf=dbuf):
                k = p * 2
                wait_g(sbuf, k, rows0, g0)
                d0 = scat(rows0, dbuf, k, s0)
                wait_g(sbuf, k + 1, rows1, g1)
                d1 = scat(rows1, dbuf, k + 1, s1)
                for d in d0:
                    d.wait()
                pltpu.async_copy(z_hbm.at[sbuf.at[k + 2]], rows0, g0)
                for d in d1:
                    d.wait()
                pltpu.async_copy(z_hbm.at[sbuf.at[k + 3]], rows1, g1)
                return 0
            lax.fori_loop(0, _G // 2 - 1, pair, 0)

            k = _G - 2
            wait_g(sbuf, k, rows0, g0)
            d0 = scat(rows0, dbuf, k, s0)
            wait_g(sbuf, k + 1, rows1, g1)
            d1 = scat(rows1, dbuf, k + 1, s1)
            if g + 1 < ngrp:
                idx_wait(g + 1)
                for d in d0:
                    d.wait()
                pltpu.async_copy(z_hbm.at[sbufs[(g + 1) % 2].at[0]], rows0, g0)
                for d in d1:
                    d.wait()
                pltpu.async_copy(z_hbm.at[sbufs[(g + 1) % 2].at[1]], rows1, g1)
            else:
                for d in d0 + d1:
                    d.wait()
        plsc.subcore_barrier()

        if with_deg:
            dd = pltpu.async_copy(
                dacc_sh.at[pl.ds(s * _DEGPT, _DEGPT)],
                deg_hbm.at[pl.ds(c * _DEGPAD + s * _DEGPT, _DEGPT)], s1)
            pltpu.sync_copy(acc_sh.at[pl.ds(base, _RPT)],
                            out_hbm.at[c, pl.ds(base, _RPT)])
            dd.wait()
        else:
            pltpu.sync_copy(acc_sh.at[pl.ds(base, _RPT)],
                            out_hbm.at[c, pl.ds(base, _RPT)])

    return spmm


_BM = 400


@functools.lru_cache(maxsize=None)
def _make_layer(n, k, f, act):
    kp = max(k, 128)
    def body(h_ref, wt_ref, wb_ref, b_ref, s_ref, d_ref, o_ref):
        dsum = d_ref[0] + d_ref[1]
        inv = 1.0 / jnp.maximum(dsum, 1.0)
        mean = (s_ref[0, :, :k] + s_ref[1, :, :k]) * inv
        r = (jnp.dot(h_ref[...], wt_ref[...],
                     preferred_element_type=jnp.float32)
             + jnp.dot(mean, wb_ref[...],
                       preferred_element_type=jnp.float32)
             + b_ref[...])
        o_ref[...] = jnp.maximum(r, 0.0) if act else r
    return pl.pallas_call(
        body,
        grid=(n // _BM,),
        in_specs=[
            pl.BlockSpec((_BM, k), lambda i: (i, 0)),
            pl.BlockSpec((k, f), lambda i: (0, 0)),
            pl.BlockSpec((k, f), lambda i: (0, 0)),
            pl.BlockSpec((1, f), lambda i: (0, 0)),
            pl.BlockSpec((_NC, _BM, kp), lambda i: (0, i, 0)),
            pl.BlockSpec((_NC, _BM, 1), lambda i: (0, i, 0)),
        ],
        out_specs=pl.BlockSpec((_BM, f), lambda i: (i, 0)),
        out_shape=jax.ShapeDtypeStruct((n, f), jnp.float32),
    )


def kernel(x, adj, W1, b1, W2, b2, W3, b3, W4, b4):
    n = x.shape[0]
    e = adj.shape[1]
    nch = e // (_NW * _CHUNK)
    src3 = adj[0].reshape(_NW, nch, _CHUNK)
    dst3 = adj[1].reshape(_NW, nch, _CHUNK)

    h = x
    deg3 = None
    for i, (W, b, act) in enumerate(((W1, b1, True), (W2, b2, True),
                                     (W3, b3, True), (W4, b4, False))):
        k = W.shape[0] // 2
        f = W.shape[1]
        if i == 0:
            s2, deg_flat = _make_spmm(n, e, k, True)(src3, dst3, h)
            deg3 = deg_flat.reshape(_NC, _DEGPAD, 1)
        else:
            s2 = _make_spmm(n, e, k, False)(src3, dst3, h)
        h = _make_layer(n, k, f, act)(
            h, W[:k], W[k:], b.reshape(1, f), s2, deg3)
    return h


# speedup vs baseline: 10.4784x; 1.2095x over previous
import functools

import jax
import jax.numpy as jnp
from jax import lax
from jax.experimental import pallas as pl
from jax.experimental.pallas import tpu as pltpu
from jax.experimental.pallas import tpu_sc as plsc

_NC = 2
_NS = 16
_NW = _NC * _NS
_CHUNK = 50
_G = 40
_NBUF = 4
_RPT = 632
_NPAD = _NS * _RPT
_DEGPT = 640
_DEGPAD = _NS * _DEGPT


def _mesh():
    return plsc.VectorSubcoreMesh(core_axis_name="c", subcore_axis_name="s")


@functools.lru_cache(maxsize=None)
def _make_spmm(n, e, f, with_deg):
    ept = e // _NW
    nch = ept // _CHUNK
    ngrp = nch // _G

    out_type = jax.ShapeDtypeStruct((_NC, _NPAD, f), jnp.float32)
    scratch = [
        pltpu.VMEM((_G, _CHUNK), jnp.int32),
        pltpu.VMEM((_G, _CHUNK), jnp.int32),
        pltpu.VMEM((_G, _CHUNK), jnp.int32),
        pltpu.VMEM((_G, _CHUNK), jnp.int32),
    ] + [pltpu.VMEM((_NBUF * _CHUNK, f), jnp.float32)
    ] + [pltpu.SemaphoreType.DMA
         for _ in range(2 * _NBUF + 2)
    ] + [pltpu.VMEM_SHARED((_NPAD, f), jnp.float32)]
    if with_deg:
        out_type = (out_type,
                    jax.ShapeDtypeStruct((_NC * _DEGPAD,), jnp.float32))
        scratch = scratch + [
            pltpu.VMEM((_DEGPT + 128,), jnp.float32),
            pltpu.VMEM_SHARED((_DEGPAD,), jnp.float32),
        ]

    @functools.partial(
        pl.kernel, out_type=out_type, mesh=_mesh(), scratch_types=scratch)
    def spmm(src3_hbm, dst3_hbm, z_hbm, *refs):
        if with_deg:
            out_hbm, deg_hbm = refs[0], refs[1]
            refs = refs[2:]
        else:
            out_hbm = refs[0]
            refs = refs[1:]
        sb0, sb1, db0, db1 = refs[0:4]
        rowsbuf = refs[4]
        rows = tuple(rowsbuf.at[pl.ds(b * _CHUNK, _CHUNK)]
                     for b in range(_NBUF))
        gsem = refs[5:5 + _NBUF]
        ssem = refs[5 + _NBUF:5 + 2 * _NBUF]
        is0, is1 = refs[5 + 2 * _NBUF:7 + 2 * _NBUF]
        acc_sh = refs[7 + 2 * _NBUF]
        if with_deg:
            dzo_v, dacc_sh = refs[8 + 2 * _NBUF:]
        c = lax.axis_index("c")
        s = lax.axis_index("s")
        wid = s * _NC + c
        sbufs = (sb0, sb1)
        dbufs = (db0, db1)
        isems = (is0, is1)

        def idx_start(g):
            pltpu.async_copy(src3_hbm.at[wid, pl.ds(g * _G, _G)],
                             sbufs[g % 2], isems[g % 2])
            pltpu.async_copy(dst3_hbm.at[wid, pl.ds(g * _G, _G)],
                             dbufs[g % 2], isems[g % 2])

        def idx_wait(g):
            pltpu.make_async_copy(src3_hbm.at[wid, pl.ds(g * _G, _G)],
                                  sbufs[g % 2], isems[g % 2]).wait()
            pltpu.make_async_copy(dst3_hbm.at[wid, pl.ds(g * _G, _G)],
                                  dbufs[g % 2], isems[g % 2]).wait()

        idx_start(0)

        def zstore(i, _):
            rowsbuf[i // (f // 16), pl.ds((i % (f // 16)) * 16, 16)] = (
                jnp.zeros((16,), jnp.float32))
            return 0
        lax.fori_loop(0, _CHUNK * (f // 16), zstore, 0)
        base = s * _RPT
        zds = []
        for r in range(_RPT // _CHUNK):
            zds.append(pltpu.async_copy(
                rowsbuf.at[pl.ds(0, _CHUNK)],
                acc_sh.at[pl.ds(base + r * _CHUNK, _CHUNK)], ssem[0]))
        rem = _RPT - (_RPT // _CHUNK) * _CHUNK
        if rem:
            zds.append(pltpu.async_copy(
                rowsbuf.at[pl.ds(0, rem)],
                acc_sh.at[pl.ds(base + (_RPT // _CHUNK) * _CHUNK, rem)],
                ssem[0]))
        if with_deg:
            for i in range(128 // 16):
                dzo_v[pl.ds(_DEGPT + i * 16, 16)] = jnp.ones((16,),
                                                             jnp.float32)

            def dzstore(i, _):
                dzo_v[pl.ds(i * 16, 16)] = jnp.zeros((16,), jnp.float32)
                return 0
            lax.fori_loop(0, _DEGPT // 16, dzstore, 0)
            zds.append(pltpu.async_copy(
                dzo_v.at[pl.ds(0, _DEGPT)],
                dacc_sh.at[pl.ds(s * _DEGPT, _DEGPT)], ssem[1]))
        for d in zds:
            d.wait()
        plsc.subcore_barrier()

        def wait_g(sbuf, k, buf, sem):
            pltpu.make_async_copy(z_hbm.at[sbuf.at[k]], buf, sem).wait()

        def scat(buf, dbuf, k, sem):
            d = pltpu.async_copy(buf, acc_sh.at[dbuf.at[k]], sem, add=True)
            if with_deg:
                dd = pltpu.async_copy(dzo_v.at[pl.ds(_DEGPT, _CHUNK)],
                                      dacc_sh.at[dbuf.at[k]], sem, add=True)
                return (d, dd)
            return (d,)

        idx_wait(0)
        for b in range(_NBUF):
            pltpu.async_copy(z_hbm.at[sb0.at[b]], rows[b], gsem[b])

        for g in range(ngrp):
            sbuf, dbuf = sbufs[g % 2], dbufs[g % 2]
            if g + 1 < ngrp:
                idx_start(g + 1)

            def wave(w, _, sbuf=sbuf, dbuf=dbuf):
                k = w * _NBUF
                ds = []
                for b in range(_NBUF):
                    wait_g(sbuf, k + b, rows[b], gsem[b])
                    ds.append(scat(rows[b], dbuf, k + b, ssem[b]))
                for b in range(_NBUF):
                    for d in ds[b]:
                        d.wait()
                    pltpu.async_copy(z_hbm.at[sbuf.at[k + _NBUF + b]],
                                     rows[b], gsem[b])
                return 0
            lax.fori_loop(0, _G // _NBUF - 1, wave, 0)

            k = _G - _NBUF
            ds = []
            for b in range(_NBUF):
                wait_g(sbuf, k + b, rows[b], gsem[b])
                ds.append(scat(rows[b], dbuf, k + b, ssem[b]))
            if g + 1 < ngrp:
                idx_wait(g + 1)
                for b in range(_NBUF):
                    for d in ds[b]:
                        d.wait()
                    pltpu.async_copy(z_hbm.at[sbufs[(g + 1) % 2].at[b]],
                                     rows[b], gsem[b])
            else:
                for dd in ds:
                    for d in dd:
                        d.wait()
        plsc.subcore_barrier()

        if with_deg:
            dd = pltpu.async_copy(
                dacc_sh.at[pl.ds(s * _DEGPT, _DEGPT)],
                deg_hbm.at[pl.ds(c * _DEGPAD + s * _DEGPT, _DEGPT)], ssem[1])
            pltpu.sync_copy(acc_sh.at[pl.ds(base, _RPT)],
                            out_hbm.at[c, pl.ds(base, _RPT)])
            dd.wait()
        else:
            pltpu.sync_copy(acc_sh.at[pl.ds(base, _RPT)],
                            out_hbm.at[c, pl.ds(base, _RPT)])

    return spmm


_BM = 400


@functools.lru_cache(maxsize=None)
def _make_layer(n, k, f, act):
    kp = max(k, 128)
    def body(h_ref, wt_ref, wb_ref, b_ref, s_ref, d_ref, o_ref):
        dsum = d_ref[0] + d_ref[1]
        inv = 1.0 / jnp.maximum(dsum, 1.0)
        mean = (s_ref[0, :, :k] + s_ref[1, :, :k]) * inv
        r = (jnp.dot(h_ref[...], wt_ref[...],
                     preferred_element_type=jnp.float32)
             + jnp.dot(mean, wb_ref[...],
                       preferred_element_type=jnp.float32)
             + b_ref[...])
        o_ref[...] = jnp.maximum(r, 0.0) if act else r
    return pl.pallas_call(
        body,
        grid=(n // _BM,),
        in_specs=[
            pl.BlockSpec((_BM, k), lambda i: (i, 0)),
            pl.BlockSpec((k, f), lambda i: (0, 0)),
            pl.BlockSpec((k, f), lambda i: (0, 0)),
            pl.BlockSpec((1, f), lambda i: (0, 0)),
            pl.BlockSpec((_NC, _BM, kp), lambda i: (0, i, 0)),
            pl.BlockSpec((_NC, _BM, 1), lambda i: (0, i, 0)),
        ],
        out_specs=pl.BlockSpec((_BM, f), lambda i: (i, 0)),
        out_shape=jax.ShapeDtypeStruct((n, f), jnp.float32),
    )


def kernel(x, adj, W1, b1, W2, b2, W3, b3, W4, b4):
    n = x.shape[0]
    e = adj.shape[1]
    nch = e // (_NW * _CHUNK)
    src3 = adj[0].reshape(_NW, nch, _CHUNK)
    dst3 = adj[1].reshape(_NW, nch, _CHUNK)

    h = x
    deg3 = None
    for i, (W, b, act) in enumerate(((W1, b1, True), (W2, b2, True),
                                     (W3, b3, True), (W4, b4, False))):
        k = W.shape[0] // 2
        f = W.shape[1]
        if i == 0:
            s2, deg_flat = _make_spmm(n, e, k, True)(src3, dst3, h)
            deg3 = deg_flat.reshape(_NC, _DEGPAD, 1)
        else:
            s2 = _make_spmm(n, e, k, False)(src3, dst3, h)
        h = _make_layer(n, k, f, act)(
            h, W[:k], W[k:], b.reshape(1, f), s2, deg3)
    return h
